# Initial kernel scaffold; baseline (speedup 1.0000x reference)
import jax
import jax.numpy as jnp
from jax.experimental import pallas as pl


def kernel(idx, targets, table):
    raise NotImplementedError("write your pallas kernel here")


# speedup vs baseline: 1.4270x; 1.4270x over previous
import functools

import jax
import jax.numpy as jnp
from jax import lax
from jax.experimental import pallas as pl
from jax.experimental.pallas import tpu as pltpu
from jax.experimental.pallas import tpu_sc as plsc

B, T, V = 1024, 50, 1000
N = B * T
NC, NS, L = 2, 16, 16
NW = NC * NS
PER_W = N // NW
K = 32
NCHUNK = PER_W // K


def _lse_body(tab_ref, out_ref):
    x = tab_ref[...]
    m = jnp.max(x, axis=1)
    s = jnp.sum(jnp.exp(x - m[:, None]), axis=1)
    out_ref[...] = jnp.log(s) + m


def _row_lse(table):
    return pl.pallas_call(
        _lse_body,
        out_shape=jax.ShapeDtypeStruct((V,), jnp.float32),
    )(table)


def _sc_body(table_hbm, idx2_hbm, tgt2_hbm, lse_hbm,
             out_hbm, lpart_hbm,
             idx2_v, tgt2_v, lse_v, rows0, rows1, acc_v,
             sem0, sem1):
    wid = lax.axis_index("s") * NC + lax.axis_index("c")
    base = wid * PER_W

    pltpu.sync_copy(idx2_hbm.at[wid], idx2_v)
    pltpu.sync_copy(tgt2_hbm.at[wid], tgt2_v)
    pltpu.sync_copy(lse_hbm, lse_v)
    acc_v[...] = jnp.zeros((L,), jnp.float32)

    def gather_start(g, buf, sem):
        pltpu.make_async_copy(table_hbm.at[idx2_v.at[g]], buf, sem).start()

    def do_chunk(g, buf, sem):
        pltpu.make_async_copy(table_hbm.at[idx2_v.at[g]], buf, sem).wait()
        for j in range(K // L):
            rv = idx2_v[g, pl.ds(j * L, L)]
            tv = tgt2_v[g, pl.ds(j * L, L)]
            lvals = plsc.load_gather(lse_v, [rv])
            lrows = lax.broadcasted_iota(jnp.int32, (L,), 0) + jnp.int32(j * L)
            picked = plsc.load_gather(buf, [lrows, tv])
            acc_v[...] = acc_v[...] + (lvals - picked)
        pltpu.sync_copy(buf, out_hbm.at[pl.ds(base + g * K, K)])

    gather_start(0, rows0, sem0)

    def ring(i, carry):
        g0 = i * 2
        gather_start(g0 + 1, rows1, sem1)
        do_chunk(g0, rows0, sem0)

        @pl.when(g0 + 2 < NCHUNK)
        def _():
            gather_start(g0 + 2, rows0, sem0)

        do_chunk(g0 + 1, rows1, sem1)
        return carry

    lax.fori_loop(0, NCHUNK // 2, ring, 0)

    pltpu.sync_copy(acc_v, lpart_hbm.at[wid])


@functools.partial(
    pl.kernel,
    out_type=(
        jax.ShapeDtypeStruct((N, V), jnp.float32),
        jax.ShapeDtypeStruct((NW, L), jnp.float32),
    ),
    mesh=plsc.VectorSubcoreMesh(core_axis_name="c", subcore_axis_name="s",
                                num_cores=NC, num_subcores=NS),
    compiler_params=pltpu.CompilerParams(use_tc_tiling_on_sc=False,
                                         needs_layout_passes=False),
    scratch_types=[
        pltpu.VMEM((NCHUNK, K), jnp.int32),
        pltpu.VMEM((NCHUNK, K), jnp.int32),
        pltpu.VMEM((1024,), jnp.float32),
        pltpu.VMEM((K, V), jnp.float32),
        pltpu.VMEM((K, V), jnp.float32),
        pltpu.VMEM((L,), jnp.float32),
        pltpu.SemaphoreType.DMA,
        pltpu.SemaphoreType.DMA,
    ],
)
def _sc_gather_loss(table_hbm, idx2_hbm, tgt2_hbm, lse_hbm,
                    out_hbm, lpart_hbm,
                    idx2_v, tgt2_v, lse_v, rows0, rows1, acc_v,
                    sem0, sem1):
    _sc_body(table_hbm, idx2_hbm, tgt2_hbm, lse_hbm,
             out_hbm, lpart_hbm,
             idx2_v, tgt2_v, lse_v, rows0, rows1, acc_v,
             sem0, sem1)


def kernel(idx, targets, table):
    idx_flat = idx.reshape(N)
    tgt_flat = targets.reshape(N)
    lse = jnp.pad(_row_lse(table), (0, 1024 - V))
    out, lpart = _sc_gather_loss(
        table,
        idx_flat.reshape(NW, NCHUNK, K),
        tgt_flat.reshape(NW, NCHUNK, K),
        lse,
    )
    loss = jnp.sum(lpart) / jnp.float32(N)
    return (out.reshape(B, T, V), loss)


# speedup vs baseline: 1.4272x; 1.0001x over previous
import functools

import jax
import jax.numpy as jnp
from jax import lax
from jax.experimental import pallas as pl
from jax.experimental.pallas import tpu as pltpu
from jax.experimental.pallas import tpu_sc as plsc

B, T, V = 1024, 50, 1000
N = B * T
NC, NS, L = 2, 16, 16
NW = NC * NS
ROWS_W = B // NW


def _lse_body(tab_ref, out_ref):
    x = tab_ref[...]
    m = jnp.max(x, axis=1)
    s = jnp.sum(jnp.exp(x - m[:, None]), axis=1)
    out_ref[...] = jnp.log(s) + m


def _row_lse(table):
    return pl.pallas_call(
        _lse_body,
        out_shape=jax.ShapeDtypeStruct((V,), jnp.float32),
    )(table)


_WINDOWS = ((0, 0), (16, 0), (32, 0), (34, 14))


def _sc_body(table_hbm, idx_hbm, tgt_hbm, lse_hbm,
             out_hbm, lpart_hbm,
             idx_v, tgt_v, lse_v, rows0, rows1, acc_v,
             sem0, sem1):
    wid = lax.axis_index("s") * NC + lax.axis_index("c")
    base = wid * ROWS_W

    pltpu.sync_copy(idx_hbm.at[pl.ds(base, ROWS_W)], idx_v)
    pltpu.sync_copy(tgt_hbm.at[pl.ds(base, ROWS_W)], tgt_v)
    pltpu.sync_copy(lse_hbm, lse_v)
    acc_v[...] = jnp.zeros((L,), jnp.float32)

    lane = lax.broadcasted_iota(jnp.int32, (L,), 0)

    def gather_start(g, buf, sem):
        pltpu.make_async_copy(table_hbm.at[idx_v.at[g]], buf, sem).start()

    def do_chunk(g, buf, sem):
        pltpu.make_async_copy(table_hbm.at[idx_v.at[g]], buf, sem).wait()
        for w, first in _WINDOWS:
            rv = idx_v[g, pl.ds(w, L)]
            tv = tgt_v[g, pl.ds(w, L)]
            lvals = plsc.load_gather(lse_v, [rv])
            picked = plsc.load_gather(buf, [lane + jnp.int32(w), tv])
            contrib = lvals - picked
            if first:
                contrib = jnp.where(lane >= jnp.int32(first), contrib, 0.0)
            acc_v[...] = acc_v[...] + contrib
        pltpu.sync_copy(buf, out_hbm.at[base + g])

    gather_start(0, rows0, sem0)

    def ring(i, carry):
        g0 = i * 2
        gather_start(g0 + 1, rows1, sem1)
        do_chunk(g0, rows0, sem0)

        @pl.when(g0 + 2 < ROWS_W)
        def _():
            gather_start(g0 + 2, rows0, sem0)

        do_chunk(g0 + 1, rows1, sem1)
        return carry

    lax.fori_loop(0, ROWS_W // 2, ring, 0)

    pltpu.sync_copy(acc_v, lpart_hbm.at[wid])


@functools.partial(
    pl.kernel,
    out_type=(
        jax.ShapeDtypeStruct((B, T, V), jnp.float32),
        jax.ShapeDtypeStruct((NW, L), jnp.float32),
    ),
    mesh=plsc.VectorSubcoreMesh(core_axis_name="c", subcore_axis_name="s",
                                num_cores=NC, num_subcores=NS),
    compiler_params=pltpu.CompilerParams(use_tc_tiling_on_sc=False,
                                         needs_layout_passes=False),
    scratch_types=[
        pltpu.VMEM((ROWS_W, T), jnp.int32),
        pltpu.VMEM((ROWS_W, T), jnp.int32),
        pltpu.VMEM((1024,), jnp.float32),
        pltpu.VMEM((T, V), jnp.float32),
        pltpu.VMEM((T, V), jnp.float32),
        pltpu.VMEM((L,), jnp.float32),
        pltpu.SemaphoreType.DMA,
        pltpu.SemaphoreType.DMA,
    ],
)
def _sc_gather_loss(table_hbm, idx_hbm, tgt_hbm, lse_hbm,
                    out_hbm, lpart_hbm,
                    idx_v, tgt_v, lse_v, rows0, rows1, acc_v,
                    sem0, sem1):
    _sc_body(table_hbm, idx_hbm, tgt_hbm, lse_hbm,
             out_hbm, lpart_hbm,
             idx_v, tgt_v, lse_v, rows0, rows1, acc_v,
             sem0, sem1)


def kernel(idx, targets, table):
    lse = jnp.pad(_row_lse(table), (0, 1024 - V))
    out, lpart = _sc_gather_loss(table, idx, targets, lse)
    loss = jnp.sum(lpart) / jnp.float32(N)
    return (out, loss)


# speedup vs baseline: 1.4564x; 1.0205x over previous
import functools

import jax
import jax.numpy as jnp
from jax import lax
from jax.experimental import pallas as pl
from jax.experimental.pallas import tpu as pltpu
from jax.experimental.pallas import tpu_sc as plsc

B, T, V = 1024, 50, 1000
N = B * T
VP = 1024
NC, NS, L = 2, 16, 16
NW = NC * NS
ROWS_W = B // NW
BB = 4

_WINDOWS = ((0, 0), (16, 0), (32, 0), (34, 14))


def _m_body(tab_ref, out_ref):
    x = tab_ref[...]
    m = jnp.max(x, axis=1)
    lse = jnp.log(jnp.sum(jnp.exp(x - m[:, None]), axis=1)) + m
    out_ref[...] = lse[:, None] - x


def _loss_table(table):
    return pl.pallas_call(
        _m_body,
        out_shape=jax.ShapeDtypeStruct((V, V), jnp.float32),
    )(table)


def _gather_body(table_hbm, idx_hbm, out_hbm, idx_v, rows0, rows1,
                 sem0, sem1):
    wid = lax.axis_index("s") * NC + lax.axis_index("c")
    base = wid * ROWS_W

    pltpu.sync_copy(idx_hbm.at[pl.ds(base, ROWS_W)], idx_v)

    def gather_start(g, buf, sem):
        pltpu.make_async_copy(table_hbm.at[idx_v.at[g]], buf, sem).start()

    def do_chunk(g, buf, sem):
        pltpu.make_async_copy(table_hbm.at[idx_v.at[g]], buf, sem).wait()
        pltpu.sync_copy(buf, out_hbm.at[pl.ds((base + g) * T, T)])

    gather_start(0, rows0, sem0)

    def ring(i, carry):
        g0 = i * 2
        gather_start(g0 + 1, rows1, sem1)
        do_chunk(g0, rows0, sem0)

        @pl.when(g0 + 2 < ROWS_W)
        def _():
            gather_start(g0 + 2, rows0, sem0)

        do_chunk(g0 + 1, rows1, sem1)
        return carry

    lax.fori_loop(0, ROWS_W // 2, ring, 0)


_sc_gather = functools.partial(
    pl.kernel,
    out_type=jax.ShapeDtypeStruct((N, 8, 128), jnp.float32),
    mesh=plsc.VectorSubcoreMesh(core_axis_name="c", subcore_axis_name="s",
                                num_cores=NC, num_subcores=NS),
    compiler_params=pltpu.CompilerParams(use_tc_tiling_on_sc=True),
    scratch_types=[
        pltpu.VMEM((ROWS_W, T), jnp.int32),
        pltpu.VMEM((T, 8, 128), jnp.float32),
        pltpu.VMEM((T, 8, 128), jnp.float32),
        pltpu.SemaphoreType.DMA,
        pltpu.SemaphoreType.DMA,
    ],
)(_gather_body)


def _relayout_body(x_ref, out_ref):
    for b in range(BB):
        x = x_ref[pl.ds(b * T, T)]
        for c in range(7):
            out_ref[b, :, pl.ds(c * 128, 128)] = x[:, c, :]
        out_ref[b, :, pl.ds(896, V - 896)] = x[:, 7, : V - 896]


def _relayout(x):
    return pl.pallas_call(
        _relayout_body,
        grid=(B // BB,),
        in_specs=[pl.BlockSpec((BB * T, 8, 128), lambda i: (i, 0, 0))],
        out_specs=pl.BlockSpec((BB, T, V), lambda i: (i, 0, 0)),
        out_shape=jax.ShapeDtypeStruct((B, T, V), jnp.float32),
    )(x)


def _loss_body(m_hbm, idx_hbm, tgt_hbm, lpart_hbm,
               idx_v, tgt_v, flat_v, mval_v, acc_v, sem):
    wid = lax.axis_index("s") * NC + lax.axis_index("c")
    base = wid * ROWS_W

    pltpu.sync_copy(idx_hbm.at[pl.ds(base, ROWS_W)], idx_v)
    pltpu.sync_copy(tgt_hbm.at[pl.ds(base, ROWS_W)], tgt_v)

    def flatten(g, carry):
        for w, _ in _WINDOWS:
            rv = idx_v[g, pl.ds(w, L)]
            tv = tgt_v[g, pl.ds(w, L)]
            flat_v[g, pl.ds(w, L)] = rv * jnp.int32(V) + tv
        return carry

    lax.fori_loop(0, ROWS_W, flatten, 0)

    def fire(g, carry):
        pltpu.make_async_copy(m_hbm.at[flat_v.at[g]], mval_v.at[g], sem).start()
        return carry

    lax.fori_loop(0, ROWS_W, fire, 0)

    acc_v[...] = jnp.zeros((L,), jnp.float32)
    lane = lax.broadcasted_iota(jnp.int32, (L,), 0)

    def drain(g, carry):
        pltpu.make_async_copy(m_hbm.at[flat_v.at[g]], mval_v.at[g], sem).wait()
        for w, first in _WINDOWS:
            vals = mval_v[g, pl.ds(w, L)]
            if first:
                vals = jnp.where(lane >= jnp.int32(first), vals, 0.0)
            acc_v[...] = acc_v[...] + vals
        return carry

    lax.fori_loop(0, ROWS_W, drain, 0)

    pltpu.sync_copy(acc_v, lpart_hbm.at[wid])


_sc_loss = functools.partial(
    pl.kernel,
    out_type=jax.ShapeDtypeStruct((NW, L), jnp.float32),
    mesh=plsc.VectorSubcoreMesh(core_axis_name="c", subcore_axis_name="s",
                                num_cores=NC, num_subcores=NS),
    compiler_params=pltpu.CompilerParams(use_tc_tiling_on_sc=False),
    scratch_types=[
        pltpu.VMEM((ROWS_W, T), jnp.int32),
        pltpu.VMEM((ROWS_W, T), jnp.int32),
        pltpu.VMEM((ROWS_W, T), jnp.int32),
        pltpu.VMEM((ROWS_W, T), jnp.float32),
        pltpu.VMEM((L,), jnp.float32),
        pltpu.SemaphoreType.DMA,
    ],
)(_loss_body)


def kernel(idx, targets, table):
    table_pad = jnp.pad(table, ((0, 0), (0, VP - V))).reshape(V, 8, 128)
    x = _sc_gather(table_pad, idx)
    logits = _relayout(x)
    m_flat = _loss_table(table).reshape(V * V)
    lpart = _sc_loss(m_flat, idx, targets)
    loss = jnp.sum(lpart) / jnp.float32(N)
    return (logits, loss)
